# baseline (speedup 1.0000x reference)
import functools

import jax
import jax.numpy as jnp
from jax import lax
from jax.experimental import pallas as pl
from jax.experimental.pallas import tpu as pltpu
from jax.experimental.pallas import tpu_sc as plsc

_WORLD = 8
_KC = 1600
_LANES = 16


def kernel(x, emb_weight):
    B, L = x.shape
    E, D = emb_weight.shape
    C = D * _WORLD
    BL = B * L

    xf = x.reshape(BL)
    tab = jnp.tile(emb_weight, (1, _WORLD)).reshape(E * C)

    info = plsc.get_sparse_core_info()
    NC, NS = info.num_cores, info.num_subcores
    NW = NC * NS
    n_w = BL // NW
    n_chunks = n_w // _KC
    groups = _KC // _LANES

    mesh = plsc.VectorSubcoreMesh(core_axis_name="c", subcore_axis_name="s")

    @functools.partial(
        pl.kernel,
        out_type=jax.ShapeDtypeStruct((BL, C), jnp.float32),
        mesh=mesh,
        compiler_params=pltpu.CompilerParams(
            needs_layout_passes=False, use_tc_tiling_on_sc=False),
        scratch_types=[
            pltpu.VMEM((E * C,), jnp.float32),
            pltpu.VMEM((_KC,), jnp.int32),
            pltpu.VMEM((_KC, C), jnp.float32),
            pltpu.VMEM((_KC, C), jnp.float32),
            pltpu.SemaphoreType.DMA,
            pltpu.SemaphoreType.DMA,
        ],
    )
    def sc_kernel(x_hbm, tab_hbm, out_hbm, tab_v, idx_v, st0, st1, sem0, sem1):
        wid = lax.axis_index("s") * NC + lax.axis_index("c")
        wbase = wid * n_w
        pltpu.sync_copy(tab_hbm, tab_v)
        iota = lax.iota(jnp.int32, _LANES)
        sts = (st0, st1)
        sems = (sem0, sem1)

        def fill(chunk, st):
            row0 = wbase + chunk * _KC
            pltpu.sync_copy(x_hbm.at[pl.ds(row0, _KC)], idx_v)

            def group(g, carry):
                xv = idx_v[pl.ds(g * _LANES, _LANES)]
                gbase = xv * C
                rowv = g * _LANES + iota
                for c in range(C):
                    vals = plsc.load_gather(tab_v, [gbase + c])
                    colv = jnp.full((_LANES,), c, jnp.int32)
                    plsc.store_scatter(st, [rowv, colv], vals)
                return carry

            lax.fori_loop(0, groups, group, 0)

        def fire(chunk, st, sem):
            row0 = wbase + chunk * _KC
            pltpu.make_async_copy(st, out_hbm.at[pl.ds(row0, _KC)], sem).start()

        def drain(chunk, st, sem):
            row0 = wbase + chunk * _KC
            pltpu.make_async_copy(st, out_hbm.at[pl.ds(row0, _KC)], sem).wait()

        for b in range(2):
            fill(b, sts[b])
            fire(b, sts[b], sems[b])

        def outer(i, carry):
            k0 = 2 * i
            for b in range(2):
                ch = k0 + b
                drain(ch, sts[b], sems[b])
                fill(ch + 2, sts[b])
                fire(ch + 2, sts[b], sems[b])
            return carry

        lax.fori_loop(0, (n_chunks - 2) // 2, outer, 0)
        for b in range(2):
            drain(n_chunks - 2 + b, sts[b], sems[b])

    out = sc_kernel(xf, tab)
    return out.reshape(B, L, C)
